# Initial kernel scaffold; baseline (speedup 1.0000x reference)
import jax
import jax.numpy as jnp
import numpy as np
from jax import lax
from jax.experimental import pallas as pl
from jax.experimental.pallas import tpu as pltpu
from jax.experimental.pallas import tpu_sc as plsc

_B = 16384
_EMB = 64
_BN_EPS = 1e-5
_GLOBAL_MEAN = 3.5

_NC = 2
_NS = 16
_NW = _NC * _NS
_BPW = _B // _NW

_BLK = 2048


def _sc_gather_body(uidx_hbm, iidx_hbm, uemb_hbm, iemb_hbm, ubias_hbm,
                    ibias_hbm, kvec_hbm,
                    ui_out, bias_out,
                    uidx_v, iidx_v, urows_v, irows_v, ub_v, ib_v, kv_v, sem):
    wid = lax.axis_index("s") * _NC + lax.axis_index("c")
    base = wid * _BPW
    pltpu.sync_copy(uidx_hbm.at[pl.ds(base, _BPW)], uidx_v)
    pltpu.sync_copy(iidx_hbm.at[pl.ds(base, _BPW)], iidx_v)
    pltpu.sync_copy(kvec_hbm, kv_v)
    c1 = pltpu.async_copy(uemb_hbm.at[uidx_v], urows_v, sem)
    c2 = pltpu.async_copy(iemb_hbm.at[iidx_v], irows_v, sem)
    c3 = pltpu.async_copy(ubias_hbm.at[uidx_v], ub_v, sem)
    c4 = pltpu.async_copy(ibias_hbm.at[iidx_v], ib_v, sem)
    c1.wait()
    c2.wait()
    c3.wait()
    c4.wait()
    kv = kv_v[...]
    for j in range(_BPW // 16):
        sl = pl.ds(j * 16, 16)
        ub_v[sl] = ub_v[sl] + ib_v[sl] + kv
    pltpu.sync_copy(urows_v, ui_out.at[pl.ds(base, _BPW), pl.ds(0, _EMB)])
    pltpu.sync_copy(irows_v, ui_out.at[pl.ds(base, _BPW), pl.ds(_EMB, _EMB)])
    pltpu.sync_copy(ub_v, bias_out.at[pl.ds(base, _BPW)])


def _sc_gather(user_idx, item_idx, user_emb, item_emb, ubias1d, ibias1d, kvec):
    fn = pl.kernel(
        _sc_gather_body,
        out_type=[
            jax.ShapeDtypeStruct((_B, 2 * _EMB), jnp.float32),
            jax.ShapeDtypeStruct((_B,), jnp.float32),
        ],
        mesh=plsc.VectorSubcoreMesh(core_axis_name="c", subcore_axis_name="s"),
        scratch_types=[
            pltpu.VMEM((_BPW,), jnp.int32),
            pltpu.VMEM((_BPW,), jnp.int32),
            pltpu.VMEM((_BPW, _EMB), jnp.float32),
            pltpu.VMEM((_BPW, _EMB), jnp.float32),
            pltpu.VMEM((_BPW,), jnp.float32),
            pltpu.VMEM((_BPW,), jnp.float32),
            pltpu.VMEM((16,), jnp.float32),
            pltpu.SemaphoreType.DMA,
        ],
    )
    return fn(user_idx, item_idx, user_emb, item_emb, ubias1d, ibias1d, kvec)


def _tc_body(ui_ref, gtc_ref, wide_ref, bias_ref, w1a_ref, w1b_ref, c1_ref,
             w2_ref, c2_ref, wout_ref, wrow_ref, out_ref):
    h1 = jnp.dot(ui_ref[...], w1a_ref[...], preferred_element_type=jnp.float32)
    h1 = h1 + jnp.dot(gtc_ref[...], w1b_ref[...],
                      preferred_element_type=jnp.float32)
    h1 = jnp.maximum(h1 + c1_ref[...], 0.0)
    h2 = jnp.dot(h1, w2_ref[...], preferred_element_type=jnp.float32)
    h2 = jnp.maximum(h2 + c2_ref[...], 0.0)
    deep = jnp.sum(h2 * wout_ref[...], axis=1)
    wide = jnp.sum(wide_ref[...] * wrow_ref[...], axis=1)
    out_ref[...] = bias_ref[...] + deep + wide


def _tc_dense(ui, gtc, wide_features, bias, w1a, w1b, c1, w2, c2, wout, wrow):
    grid = (_B // _BLK,)
    return pl.pallas_call(
        _tc_body,
        grid=grid,
        in_specs=[
            pl.BlockSpec((_BLK, 2 * _EMB), lambda i: (i, 0)),
            pl.BlockSpec((_BLK, 128), lambda i: (i, 0)),
            pl.BlockSpec((_BLK, 36), lambda i: (i, 0)),
            pl.BlockSpec((_BLK,), lambda i: (i,)),
            pl.BlockSpec((2 * _EMB, 256), lambda i: (0, 0)),
            pl.BlockSpec((128, 256), lambda i: (0, 0)),
            pl.BlockSpec((1, 256), lambda i: (0, 0)),
            pl.BlockSpec((256, 128), lambda i: (0, 0)),
            pl.BlockSpec((1, 128), lambda i: (0, 0)),
            pl.BlockSpec((1, 128), lambda i: (0, 0)),
            pl.BlockSpec((1, 36), lambda i: (0, 0)),
        ],
        out_specs=pl.BlockSpec((_BLK,), lambda i: (i,)),
        out_shape=jax.ShapeDtypeStruct((_B,), jnp.float32),
        compiler_params=pltpu.CompilerParams(
            dimension_semantics=("arbitrary",),
        ),
    )(ui, gtc, wide_features, bias, w1a, w1b, c1, w2, c2, wout, wrow)


def kernel(user_idx, item_idx, genre, tag, wide_features, deep_continuous,
           user_bias, item_bias, user_emb, item_emb, wide_W, wide_b,
           W1, b1, g1, be1, W2, b2, g2, be2, Wout, bout):
    inv = np.float32(1.0 / np.sqrt(1.0 + _BN_EPS))
    s1 = g1 * inv
    s2 = g2 * inv
    w1f = (W1 * s1[:, None]).T
    c1 = (b1 * s1 + be1)[None, :]
    w2f = (W2 * s2[:, None]).T
    c2 = (b2 * s2 + be2)[None, :]
    w1a = w1f[: 2 * _EMB]
    w1b = jnp.pad(w1f[2 * _EMB:], ((0, 3), (0, 0)))
    gtc = jnp.concatenate(
        [genre, tag, deep_continuous,
         jnp.zeros((_B, 3), jnp.float32)], axis=1)
    kvec = jnp.broadcast_to(
        wide_b[0] + bout[0] + jnp.float32(_GLOBAL_MEAN), (16,))
    ui, bias = _sc_gather(user_idx, item_idx, user_emb, item_emb,
                          user_bias[:, 0], item_bias[:, 0], kvec)
    return _tc_dense(ui, gtc, wide_features, bias,
                     w1a, w1b, c1, w2f, c2, Wout, wide_W)


# speedup vs baseline: 1.3649x; 1.3649x over previous
import jax
import jax.numpy as jnp
import numpy as np
from jax import lax
from jax.experimental import pallas as pl
from jax.experimental.pallas import tpu as pltpu
from jax.experimental.pallas import tpu_sc as plsc

_B = 16384
_EMB = 64
_BN_EPS = 1e-5
_GLOBAL_MEAN = 3.5

_NC = 2
_NS = 16
_NW = _NC * _NS
_BPW = _B // _NW

_BLK = 2048


def _sc_gather_body(uidx_hbm, iidx_hbm, uemb2_hbm, iemb2_hbm, ubias_hbm,
                    ibias_hbm, kvec_hbm,
                    u_out, i_out, bias_out,
                    uidx_v, iidx_v, pair_v, ub_v, ib_v, kv_v, sem):
    wid = lax.axis_index("s") * _NC + lax.axis_index("c")
    base = wid * _BPW
    pltpu.sync_copy(uidx_hbm.at[pl.ds(base, _BPW)], uidx_v)
    pltpu.sync_copy(iidx_hbm.at[pl.ds(base, _BPW)], iidx_v)
    pltpu.sync_copy(kvec_hbm, kv_v)
    c3 = pltpu.async_copy(ubias_hbm.at[uidx_v], ub_v, sem)
    c4 = pltpu.async_copy(ibias_hbm.at[iidx_v], ib_v, sem)
    for j in range(_BPW // 16):
        sl = pl.ds(j * 16, 16)
        uidx_v[sl] = lax.shift_right_logical(uidx_v[sl], 1)
        iidx_v[sl] = lax.shift_right_logical(iidx_v[sl], 1)
    c1 = pltpu.async_copy(uemb2_hbm.at[uidx_v], pair_v, sem)
    c3.wait()
    c4.wait()
    kv = kv_v[...]
    for j in range(_BPW // 16):
        sl = pl.ds(j * 16, 16)
        ub_v[sl] = ub_v[sl] + ib_v[sl] + kv
    pltpu.sync_copy(ub_v, bias_out.at[pl.ds(base, _BPW)])
    c1.wait()
    pltpu.sync_copy(pair_v, u_out.at[pl.ds(base, _BPW), :])
    c2 = pltpu.async_copy(iemb2_hbm.at[iidx_v], pair_v, sem)
    c2.wait()
    pltpu.sync_copy(pair_v, i_out.at[pl.ds(base, _BPW), :])


def _sc_gather(user_idx, item_idx, uemb2, iemb2, ubias1d, ibias1d, kvec):
    fn = pl.kernel(
        _sc_gather_body,
        out_type=[
            jax.ShapeDtypeStruct((_B, 2 * _EMB), jnp.float32),
            jax.ShapeDtypeStruct((_B, 2 * _EMB), jnp.float32),
            jax.ShapeDtypeStruct((_B,), jnp.float32),
        ],
        mesh=plsc.VectorSubcoreMesh(core_axis_name="c", subcore_axis_name="s"),
        scratch_types=[
            pltpu.VMEM((_BPW,), jnp.int32),
            pltpu.VMEM((_BPW,), jnp.int32),
            pltpu.VMEM((_BPW, 2 * _EMB), jnp.float32),
            pltpu.VMEM((_BPW,), jnp.float32),
            pltpu.VMEM((_BPW,), jnp.float32),
            pltpu.VMEM((16,), jnp.float32),
            pltpu.SemaphoreType.DMA,
        ],
    )
    return fn(user_idx, item_idx, uemb2, iemb2, ubias1d, ibias1d, kvec)


def _tc_body(up_ref, ip_ref, uidx_ref, iidx_ref, gtc_ref, wide_ref, bias_ref,
             w1u_ref, w1i_ref, w1b_ref, c1_ref, w2_ref, c2_ref, wout_ref,
             wrow_ref, out_ref):
    uodd = (uidx_ref[...] & 1) > 0
    iodd = (iidx_ref[...] & 1) > 0
    u = jnp.where(uodd, up_ref[:, _EMB:], up_ref[:, :_EMB])
    i = jnp.where(iodd, ip_ref[:, _EMB:], ip_ref[:, :_EMB])
    h1 = jnp.dot(u, w1u_ref[...], preferred_element_type=jnp.float32)
    h1 = h1 + jnp.dot(i, w1i_ref[...], preferred_element_type=jnp.float32)
    h1 = h1 + jnp.dot(gtc_ref[...], w1b_ref[...],
                      preferred_element_type=jnp.float32)
    h1 = jnp.maximum(h1 + c1_ref[...], 0.0)
    h2 = jnp.dot(h1, w2_ref[...], preferred_element_type=jnp.float32)
    h2 = jnp.maximum(h2 + c2_ref[...], 0.0)
    deep = jnp.sum(h2 * wout_ref[...], axis=1)
    wide = jnp.sum(wide_ref[...] * wrow_ref[...], axis=1)
    out_ref[...] = bias_ref[...] + deep + wide


def _tc_dense(up, ip, user_idx, item_idx, gtc, wide_features, bias,
              w1u, w1i, w1b, c1, w2, c2, wout, wrow):
    grid = (_B // _BLK,)
    return pl.pallas_call(
        _tc_body,
        grid=grid,
        in_specs=[
            pl.BlockSpec((_BLK, 2 * _EMB), lambda i: (i, 0)),
            pl.BlockSpec((_BLK, 2 * _EMB), lambda i: (i, 0)),
            pl.BlockSpec((_BLK, 1), lambda i: (i, 0)),
            pl.BlockSpec((_BLK, 1), lambda i: (i, 0)),
            pl.BlockSpec((_BLK, 128), lambda i: (i, 0)),
            pl.BlockSpec((_BLK, 36), lambda i: (i, 0)),
            pl.BlockSpec((_BLK,), lambda i: (i,)),
            pl.BlockSpec((_EMB, 256), lambda i: (0, 0)),
            pl.BlockSpec((_EMB, 256), lambda i: (0, 0)),
            pl.BlockSpec((128, 256), lambda i: (0, 0)),
            pl.BlockSpec((1, 256), lambda i: (0, 0)),
            pl.BlockSpec((256, 128), lambda i: (0, 0)),
            pl.BlockSpec((1, 128), lambda i: (0, 0)),
            pl.BlockSpec((1, 128), lambda i: (0, 0)),
            pl.BlockSpec((1, 36), lambda i: (0, 0)),
        ],
        out_specs=pl.BlockSpec((_BLK,), lambda i: (i,)),
        out_shape=jax.ShapeDtypeStruct((_B,), jnp.float32),
        compiler_params=pltpu.CompilerParams(
            dimension_semantics=("arbitrary",),
        ),
    )(up, ip, user_idx, item_idx, gtc, wide_features, bias,
      w1u, w1i, w1b, c1, w2, c2, wout, wrow)


def kernel(user_idx, item_idx, genre, tag, wide_features, deep_continuous,
           user_bias, item_bias, user_emb, item_emb, wide_W, wide_b,
           W1, b1, g1, be1, W2, b2, g2, be2, Wout, bout):
    n_rows = user_emb.shape[0]
    inv = np.float32(1.0 / np.sqrt(1.0 + _BN_EPS))
    s1 = g1 * inv
    s2 = g2 * inv
    w1f = (W1 * s1[:, None]).T
    c1 = (b1 * s1 + be1)[None, :]
    w2f = (W2 * s2[:, None]).T
    c2 = (b2 * s2 + be2)[None, :]
    w1u = w1f[:_EMB]
    w1i = w1f[_EMB:2 * _EMB]
    w1b = jnp.pad(w1f[2 * _EMB:], ((0, 3), (0, 0)))
    gtc = jnp.concatenate(
        [genre, tag, deep_continuous,
         jnp.zeros((_B, 3), jnp.float32)], axis=1)
    kvec = jnp.broadcast_to(
        wide_b[0] + bout[0] + jnp.float32(_GLOBAL_MEAN), (16,))
    uemb2 = user_emb.reshape(n_rows // 2, 2 * _EMB)
    iemb2 = item_emb.reshape(n_rows // 2, 2 * _EMB)
    up, ip, bias = _sc_gather(user_idx, item_idx, uemb2, iemb2,
                              user_bias[:, 0], item_bias[:, 0], kvec)
    return _tc_dense(up, ip, user_idx[:, None], item_idx[:, None],
                     gtc, wide_features, bias,
                     w1u, w1i, w1b, c1, w2f, c2, Wout, wide_W)


# speedup vs baseline: 1.4836x; 1.0869x over previous
import jax
import jax.numpy as jnp
import numpy as np
from jax import lax
from jax.experimental import pallas as pl
from jax.experimental.pallas import tpu as pltpu
from jax.experimental.pallas import tpu_sc as plsc

_B = 16384
_EMB = 64
_BN_EPS = 1e-5
_GLOBAL_MEAN = 3.5

_NC = 2
_NS = 16
_NW = _NC * _NS
_BPW = _B // _NW
_CH = 128

_BLK = 2048


def _sc_gather_body(uidx_hbm, iidx_hbm, uemb2_hbm, iemb2_hbm, ubias_hbm,
                    ibias_hbm, kvec_hbm,
                    u_out, i_out, bias_out,
                    uraw_v, iraw_v, uq_v, iq_v, uo_v, io_v,
                    p0_v, p1_v, sel_v, ub_v, ib_v, kv_v,
                    sem0, sem1, semb):
    wid = lax.axis_index("s") * _NC + lax.axis_index("c")
    base = wid * _BPW
    pltpu.sync_copy(uidx_hbm.at[pl.ds(base, _BPW)], uraw_v)
    pltpu.sync_copy(iidx_hbm.at[pl.ds(base, _BPW)], iraw_v)
    pltpu.sync_copy(kvec_hbm, kv_v)
    cb_u = pltpu.async_copy(ubias_hbm.at[uraw_v], ub_v, semb)
    cb_i = pltpu.async_copy(ibias_hbm.at[iraw_v], ib_v, semb)
    for j in range(_BPW // 16):
        sl = pl.ds(j * 16, 16)
        ur = uraw_v[sl]
        ir = iraw_v[sl]
        uq_v[sl] = lax.shift_right_logical(ur, 1)
        iq_v[sl] = lax.shift_right_logical(ir, 1)
        uo_v[sl] = lax.shift_left((ur & 1), 6)
        io_v[sl] = lax.shift_left((ir & 1), 6)

    bufs = (p0_v, p1_v)
    sems = (sem0, sem1)
    n_u = _BPW // _CH
    jobs = [(uq_v, uo_v, u_out, c) for c in range(n_u)] + \
           [(iq_v, io_v, i_out, c) for c in range(n_u)]

    def select_and_write(job_id, slot):
        _, off_ref, out_hbm, c = jobs[job_id]
        buf = bufs[slot]

        def body(r16, carry):
            offs = off_ref[pl.ds(c * _CH + r16 * 16, 16)]
            for k in range(16):
                off = offs[k]
                r = r16 * 16 + k
                for jj in range(_EMB // 16):
                    sel_v[r, pl.ds(16 * jj, 16)] = \
                        buf[r, pl.ds(off + 16 * jj, 16)]
            return carry

        lax.fori_loop(0, _CH // 16, body, 0)
        pltpu.sync_copy(sel_v,
                        out_hbm.at[pl.ds(base + c * _CH, _CH), :])

    inflight = [None, None]
    for j, (qref, _, _, c) in enumerate(jobs):
        slot = j % 2
        if inflight[slot] is not None:
            inflight[slot].wait()
            select_and_write(j - 2, slot)
        src = uemb2_hbm if j < n_u else iemb2_hbm
        inflight[slot] = pltpu.async_copy(
            src.at[qref.at[pl.ds(c * _CH, _CH)]], bufs[slot], sems[slot])
    inflight[len(jobs) % 2].wait()
    select_and_write(len(jobs) - 2, len(jobs) % 2)
    inflight[(len(jobs) + 1) % 2].wait()
    select_and_write(len(jobs) - 1, (len(jobs) + 1) % 2)

    cb_u.wait()
    cb_i.wait()
    kv = kv_v[...]
    for j in range(_BPW // 16):
        sl = pl.ds(j * 16, 16)
        ub_v[sl] = ub_v[sl] + ib_v[sl] + kv
    pltpu.sync_copy(ub_v, bias_out.at[pl.ds(base, _BPW)])


def _sc_gather(user_idx, item_idx, uemb2, iemb2, ubias1d, ibias1d, kvec):
    fn = pl.kernel(
        _sc_gather_body,
        out_type=[
            jax.ShapeDtypeStruct((_B, _EMB), jnp.float32),
            jax.ShapeDtypeStruct((_B, _EMB), jnp.float32),
            jax.ShapeDtypeStruct((_B,), jnp.float32),
        ],
        mesh=plsc.VectorSubcoreMesh(core_axis_name="c", subcore_axis_name="s"),
        scratch_types=[
            pltpu.VMEM((_BPW,), jnp.int32),
            pltpu.VMEM((_BPW,), jnp.int32),
            pltpu.VMEM((_BPW,), jnp.int32),
            pltpu.VMEM((_BPW,), jnp.int32),
            pltpu.VMEM((_BPW,), jnp.int32),
            pltpu.VMEM((_BPW,), jnp.int32),
            pltpu.VMEM((_CH, 2 * _EMB), jnp.float32),
            pltpu.VMEM((_CH, 2 * _EMB), jnp.float32),
            pltpu.VMEM((_CH, _EMB), jnp.float32),
            pltpu.VMEM((_BPW,), jnp.float32),
            pltpu.VMEM((_BPW,), jnp.float32),
            pltpu.VMEM((16,), jnp.float32),
            pltpu.SemaphoreType.DMA,
            pltpu.SemaphoreType.DMA,
            pltpu.SemaphoreType.DMA,
        ],
    )
    return fn(user_idx, item_idx, uemb2, iemb2, ubias1d, ibias1d, kvec)


def _tc_body(u_ref, i_ref, g_ref, t_ref, cn_ref, wide_ref, bias_ref,
             w1u_ref, w1i_ref, w1g_ref, w1t_ref, w1c_ref, c1_ref,
             w2_ref, c2_ref, wout_ref, wrow_ref, out_ref):
    f32 = jnp.float32
    h1 = jnp.dot(u_ref[...], w1u_ref[...], preferred_element_type=f32)
    h1 = h1 + jnp.dot(i_ref[...], w1i_ref[...], preferred_element_type=f32)
    h1 = h1 + jnp.dot(g_ref[...], w1g_ref[...], preferred_element_type=f32)
    h1 = h1 + jnp.dot(t_ref[...], w1t_ref[...], preferred_element_type=f32)
    h1 = h1 + jnp.dot(cn_ref[...], w1c_ref[...], preferred_element_type=f32)
    h1 = jnp.maximum(h1 + c1_ref[...], 0.0)
    h2 = jnp.dot(h1, w2_ref[...], preferred_element_type=f32)
    h2 = jnp.maximum(h2 + c2_ref[...], 0.0)
    deep = jnp.dot(h2, wout_ref[...], preferred_element_type=f32)
    wide = jnp.dot(wide_ref[...], wrow_ref[...], preferred_element_type=f32)
    out_ref[...] = bias_ref[...] + deep[:, 0] + wide[:, 0]


def _tc_dense(u, i, genre, tag, cont, wide_features, bias,
              w1u, w1i, w1g, w1t, w1c, c1, w2, c2, woutT, wrowT):
    grid = (_B // _BLK,)
    return pl.pallas_call(
        _tc_body,
        grid=grid,
        in_specs=[
            pl.BlockSpec((_BLK, _EMB), lambda i: (i, 0)),
            pl.BlockSpec((_BLK, _EMB), lambda i: (i, 0)),
            pl.BlockSpec((_BLK, 20), lambda i: (i, 0)),
            pl.BlockSpec((_BLK, 100), lambda i: (i, 0)),
            pl.BlockSpec((_BLK, 5), lambda i: (i, 0)),
            pl.BlockSpec((_BLK, 36), lambda i: (i, 0)),
            pl.BlockSpec((_BLK,), lambda i: (i,)),
            pl.BlockSpec((_EMB, 256), lambda i: (0, 0)),
            pl.BlockSpec((_EMB, 256), lambda i: (0, 0)),
            pl.BlockSpec((20, 256), lambda i: (0, 0)),
            pl.BlockSpec((100, 256), lambda i: (0, 0)),
            pl.BlockSpec((5, 256), lambda i: (0, 0)),
            pl.BlockSpec((1, 256), lambda i: (0, 0)),
            pl.BlockSpec((256, 128), lambda i: (0, 0)),
            pl.BlockSpec((1, 128), lambda i: (0, 0)),
            pl.BlockSpec((128, 1), lambda i: (0, 0)),
            pl.BlockSpec((36, 1), lambda i: (0, 0)),
        ],
        out_specs=pl.BlockSpec((_BLK,), lambda i: (i,)),
        out_shape=jax.ShapeDtypeStruct((_B,), jnp.float32),
        compiler_params=pltpu.CompilerParams(
            dimension_semantics=("arbitrary",),
        ),
    )(u, i, genre, tag, cont, wide_features, bias,
      w1u, w1i, w1g, w1t, w1c, c1, w2, c2, woutT, wrowT)


def kernel(user_idx, item_idx, genre, tag, wide_features, deep_continuous,
           user_bias, item_bias, user_emb, item_emb, wide_W, wide_b,
           W1, b1, g1, be1, W2, b2, g2, be2, Wout, bout):
    n_rows = user_emb.shape[0]
    inv = np.float32(1.0 / np.sqrt(1.0 + _BN_EPS))
    s1 = g1 * inv
    s2 = g2 * inv
    w1f = (W1 * s1[:, None]).T
    c1 = (b1 * s1 + be1)[None, :]
    w2f = (W2 * s2[:, None]).T
    c2 = (b2 * s2 + be2)[None, :]
    w1u = w1f[:_EMB]
    w1i = w1f[_EMB:2 * _EMB]
    w1g = w1f[128:148]
    w1t = w1f[148:248]
    w1c = w1f[248:253]
    kvec = jnp.broadcast_to(
        wide_b[0] + bout[0] + jnp.float32(_GLOBAL_MEAN), (16,))
    uemb2 = user_emb.reshape(n_rows // 2, 2 * _EMB)
    iemb2 = item_emb.reshape(n_rows // 2, 2 * _EMB)
    u, i, bias = _sc_gather(user_idx, item_idx, uemb2, iemb2,
                            user_bias[:, 0], item_bias[:, 0], kvec)
    return _tc_dense(u, i, genre, tag, deep_continuous, wide_features, bias,
                     w1u, w1i, w1g, w1t, w1c, c1, w2f, c2,
                     Wout.T, wide_W.T)


# speedup vs baseline: 1.5192x; 1.0240x over previous
import jax
import jax.numpy as jnp
import numpy as np
from jax import lax
from jax.experimental import pallas as pl
from jax.experimental.pallas import tpu as pltpu
from jax.experimental.pallas import tpu_sc as plsc

_B = 16384
_EMB = 64
_BN_EPS = 1e-5
_GLOBAL_MEAN = 3.5

_NC = 2
_NS = 16
_NW = _NC * _NS
_BPW = _B // _NW
_CH = 128

_BLK = 2048


def _sc_gather_body(uidx_hbm, iidx_hbm, uemb2_hbm, iemb2_hbm, ubias_hbm,
                    ibias_hbm, kvec_hbm,
                    u_out, i_out, bias_out,
                    uraw_v, iraw_v, uq_v, iq_v, uo_v, io_v,
                    p0_v, p1_v, sel_v, ub_v, ib_v, kv_v,
                    sem0, sem1, semb):
    wid = lax.axis_index("s") * _NC + lax.axis_index("c")
    base = wid * _BPW
    pltpu.sync_copy(uidx_hbm.at[pl.ds(base, _BPW)], uraw_v)
    pltpu.sync_copy(iidx_hbm.at[pl.ds(base, _BPW)], iraw_v)
    pltpu.sync_copy(kvec_hbm, kv_v)
    cb_u = pltpu.async_copy(ubias_hbm.at[uraw_v], ub_v, semb)
    cb_i = pltpu.async_copy(ibias_hbm.at[iraw_v], ib_v, semb)
    for j in range(_BPW // 16):
        sl = pl.ds(j * 16, 16)
        ur = uraw_v[sl]
        ir = iraw_v[sl]
        uq_v[sl] = lax.shift_right_logical(ur, 1)
        iq_v[sl] = lax.shift_right_logical(ir, 1)
        uo_v[sl] = lax.shift_left((ur & 1), 6)
        io_v[sl] = lax.shift_left((ir & 1), 6)

    bufs = (p0_v, p1_v)
    sems = (sem0, sem1)
    n_u = _BPW // _CH
    jobs = [(uq_v, uo_v, u_out, c) for c in range(n_u)] + \
           [(iq_v, io_v, i_out, c) for c in range(n_u)]

    def select_and_write(job_id, slot):
        _, off_ref, out_hbm, c = jobs[job_id]
        buf = bufs[slot]

        def body(r16, carry):
            offs = off_ref[pl.ds(c * _CH + r16 * 16, 16)]
            for k in range(16):
                off = offs[k]
                r = r16 * 16 + k
                for jj in range(_EMB // 16):
                    sel_v[r, pl.ds(16 * jj, 16)] = \
                        buf[r, pl.ds(off + 16 * jj, 16)]
            return carry

        lax.fori_loop(0, _CH // 16, body, 0)
        pltpu.sync_copy(sel_v,
                        out_hbm.at[pl.ds(base + c * _CH, _CH), :])

    inflight = [None, None]
    for j, (qref, _, _, c) in enumerate(jobs):
        slot = j % 2
        if inflight[slot] is not None:
            inflight[slot].wait()
            select_and_write(j - 2, slot)
        src = uemb2_hbm if j < n_u else iemb2_hbm
        inflight[slot] = pltpu.async_copy(
            src.at[qref.at[pl.ds(c * _CH, _CH)]], bufs[slot], sems[slot])
    inflight[len(jobs) % 2].wait()
    select_and_write(len(jobs) - 2, len(jobs) % 2)
    inflight[(len(jobs) + 1) % 2].wait()
    select_and_write(len(jobs) - 1, (len(jobs) + 1) % 2)

    cb_u.wait()
    cb_i.wait()
    kv = kv_v[...]
    for j in range(_BPW // 16):
        sl = pl.ds(j * 16, 16)
        ub_v[sl] = ub_v[sl] + ib_v[sl] + kv
    pltpu.sync_copy(ub_v, bias_out.at[pl.ds(base, _BPW)])


def _sc_gather(user_idx, item_idx, uemb2, iemb2, ubias1d, ibias1d, kvec):
    fn = pl.kernel(
        _sc_gather_body,
        out_type=[
            jax.ShapeDtypeStruct((_B, _EMB), jnp.float32),
            jax.ShapeDtypeStruct((_B, _EMB), jnp.float32),
            jax.ShapeDtypeStruct((_B,), jnp.float32),
        ],
        mesh=plsc.VectorSubcoreMesh(core_axis_name="c", subcore_axis_name="s"),
        scratch_types=[
            pltpu.VMEM((_BPW,), jnp.int32),
            pltpu.VMEM((_BPW,), jnp.int32),
            pltpu.VMEM((_BPW,), jnp.int32),
            pltpu.VMEM((_BPW,), jnp.int32),
            pltpu.VMEM((_BPW,), jnp.int32),
            pltpu.VMEM((_BPW,), jnp.int32),
            pltpu.VMEM((_CH, 2 * _EMB), jnp.float32),
            pltpu.VMEM((_CH, 2 * _EMB), jnp.float32),
            pltpu.VMEM((_CH, _EMB), jnp.float32),
            pltpu.VMEM((_BPW,), jnp.float32),
            pltpu.VMEM((_BPW,), jnp.float32),
            pltpu.VMEM((16,), jnp.float32),
            pltpu.SemaphoreType.DMA,
            pltpu.SemaphoreType.DMA,
            pltpu.SemaphoreType.DMA,
        ],
    )
    return fn(user_idx, item_idx, uemb2, iemb2, ubias1d, ibias1d, kvec)


def _tc_body(u_ref, i_ref, g_ref, t_ref, cn_ref, wide_ref, bias_ref,
             w1u_ref, w1i_ref, w1g_ref, w1t_ref, w1c_ref, c1_ref,
             w2_ref, c2_ref, wout_ref, wrow_ref, out_ref):
    f32 = jnp.float32
    tdot = lambda a, b: lax.dot_general(
        a, b, (((0,), (0,)), ((), ())), preferred_element_type=f32)
    h1 = jnp.dot(u_ref[...], w1u_ref[...], preferred_element_type=f32)
    h1 = h1 + jnp.dot(i_ref[...], w1i_ref[...], preferred_element_type=f32)
    h1 = h1 + tdot(g_ref[...], w1g_ref[...])
    h1 = h1 + tdot(t_ref[...], w1t_ref[...])
    h1 = h1 + tdot(cn_ref[...], w1c_ref[...])
    h1 = jnp.maximum(h1 + c1_ref[...], 0.0)
    h2 = jnp.dot(h1, w2_ref[...], preferred_element_type=f32)
    h2 = jnp.maximum(h2 + c2_ref[...], 0.0)
    deep = jnp.dot(h2, wout_ref[...], preferred_element_type=f32)
    wide = tdot(wide_ref[...], wrow_ref[...])
    out_ref[...] = bias_ref[...] + deep[:, 0] + wide[:, 0]


def _tc_dense(u, i, genre, tag, cont, wide_features, bias,
              w1u, w1i, w1g, w1t, w1c, c1, w2, c2, woutT, wrowT):
    grid = (_B // _BLK,)
    return pl.pallas_call(
        _tc_body,
        grid=grid,
        in_specs=[
            pl.BlockSpec((_BLK, _EMB), lambda i: (i, 0)),
            pl.BlockSpec((_BLK, _EMB), lambda i: (i, 0)),
            pl.BlockSpec((20, _BLK), lambda i: (0, i)),
            pl.BlockSpec((100, _BLK), lambda i: (0, i)),
            pl.BlockSpec((5, _BLK), lambda i: (0, i)),
            pl.BlockSpec((36, _BLK), lambda i: (0, i)),
            pl.BlockSpec((_BLK,), lambda i: (i,)),
            pl.BlockSpec((_EMB, 256), lambda i: (0, 0)),
            pl.BlockSpec((_EMB, 256), lambda i: (0, 0)),
            pl.BlockSpec((20, 256), lambda i: (0, 0)),
            pl.BlockSpec((100, 256), lambda i: (0, 0)),
            pl.BlockSpec((5, 256), lambda i: (0, 0)),
            pl.BlockSpec((1, 256), lambda i: (0, 0)),
            pl.BlockSpec((256, 128), lambda i: (0, 0)),
            pl.BlockSpec((1, 128), lambda i: (0, 0)),
            pl.BlockSpec((128, 1), lambda i: (0, 0)),
            pl.BlockSpec((36, 1), lambda i: (0, 0)),
        ],
        out_specs=pl.BlockSpec((_BLK,), lambda i: (i,)),
        out_shape=jax.ShapeDtypeStruct((_B,), jnp.float32),
        compiler_params=pltpu.CompilerParams(
            dimension_semantics=("arbitrary",),
        ),
    )(u, i, genre, tag, cont, wide_features, bias,
      w1u, w1i, w1g, w1t, w1c, c1, w2, c2, woutT, wrowT)


def kernel(user_idx, item_idx, genre, tag, wide_features, deep_continuous,
           user_bias, item_bias, user_emb, item_emb, wide_W, wide_b,
           W1, b1, g1, be1, W2, b2, g2, be2, Wout, bout):
    n_rows = user_emb.shape[0]
    inv = np.float32(1.0 / np.sqrt(1.0 + _BN_EPS))
    s1 = g1 * inv
    s2 = g2 * inv
    w1f = (W1 * s1[:, None]).T
    c1 = (b1 * s1 + be1)[None, :]
    w2f = (W2 * s2[:, None]).T
    c2 = (b2 * s2 + be2)[None, :]
    w1u = w1f[:_EMB]
    w1i = w1f[_EMB:2 * _EMB]
    w1g = w1f[128:148]
    w1t = w1f[148:248]
    w1c = w1f[248:253]
    kvec = jnp.broadcast_to(
        wide_b[0] + bout[0] + jnp.float32(_GLOBAL_MEAN), (16,))
    uemb2 = user_emb.reshape(n_rows // 2, 2 * _EMB)
    iemb2 = item_emb.reshape(n_rows // 2, 2 * _EMB)
    u, i, bias = _sc_gather(user_idx, item_idx, uemb2, iemb2,
                            user_bias[:, 0], item_bias[:, 0], kvec)
    return _tc_dense(u, i, genre.T, tag.T, deep_continuous.T,
                     wide_features.T, bias,
                     w1u, w1i, w1g, w1t, w1c, c1, w2f, c2,
                     Wout.T, wide_W.T)


# speedup vs baseline: 1.8484x; 1.2167x over previous
import jax
import jax.numpy as jnp
import numpy as np
from jax import lax
from jax.experimental import pallas as pl
from jax.experimental.pallas import tpu as pltpu
from jax.experimental.pallas import tpu_sc as plsc

_B = 16384
_EMB = 64
_BN_EPS = 1e-5
_GLOBAL_MEAN = 3.5

_NC = 2
_NS = 16
_NW = _NC * _NS
_BPW = _B // _NW
_CH = 128

_BLK = 2048


def _sc_gather_body(uidx_hbm, iidx_hbm, emb2_hbm, ubias_hbm, ibias_hbm,
                    kvec_hbm,
                    u_out, i_out, bias_out,
                    uraw_v, iraw_v, ub_v, ib_v, kv_v, p0_v, p1_v,
                    sem0, sem1, semb):
    wid = lax.axis_index("s") * _NC + lax.axis_index("c")
    base = wid * _BPW
    pltpu.sync_copy(uidx_hbm.at[pl.ds(base, _BPW)], uraw_v)
    pltpu.sync_copy(iidx_hbm.at[pl.ds(base, _BPW)], iraw_v)
    pltpu.sync_copy(kvec_hbm, kv_v)
    cb_u = pltpu.async_copy(ubias_hbm.at[uraw_v], ub_v, semb)
    cb_i = pltpu.async_copy(ibias_hbm.at[iraw_v], ib_v, semb)

    bufs = (p0_v, p1_v)
    sems = (sem0, sem1)
    n_c = _BPW // _CH
    jobs = [(uraw_v, u_out, c) for c in range(n_c)] + \
           [(iraw_v, i_out, c) for c in range(n_c)]

    inflight = [None, None]
    for j, (qref, out_hbm, c) in enumerate(jobs):
        slot = j % 2
        if inflight[slot] is not None:
            inflight[slot].wait()
            prev_out, prev_c = jobs[j - 2][1], jobs[j - 2][2]
            pltpu.sync_copy(bufs[slot],
                            prev_out.at[pl.ds(base + prev_c * _CH, _CH), :])
        inflight[slot] = pltpu.async_copy(
            emb2_hbm.at[qref.at[pl.ds(c * _CH, _CH)]], bufs[slot], sems[slot])
    for j in (len(jobs) - 2, len(jobs) - 1):
        slot = j % 2
        inflight[slot].wait()
        out_hbm, c = jobs[j][1], jobs[j][2]
        pltpu.sync_copy(bufs[slot],
                        out_hbm.at[pl.ds(base + c * _CH, _CH), :])

    cb_u.wait()
    cb_i.wait()
    kv = kv_v[...]
    for j in range(_BPW // 16):
        sl = pl.ds(j * 16, 16)
        ub_v[sl] = ub_v[sl] + ib_v[sl] + kv
    pltpu.sync_copy(ub_v, bias_out.at[pl.ds(base, _BPW)])


def _sc_gather(user_idx, item_idx, emb2, ubias1d, ibias1d, kvec):
    fn = pl.kernel(
        _sc_gather_body,
        out_type=[
            jax.ShapeDtypeStruct((_B, 2 * _EMB), jnp.float32),
            jax.ShapeDtypeStruct((_B, 2 * _EMB), jnp.float32),
            jax.ShapeDtypeStruct((_B,), jnp.float32),
        ],
        mesh=plsc.VectorSubcoreMesh(core_axis_name="c", subcore_axis_name="s"),
        scratch_types=[
            pltpu.VMEM((_BPW,), jnp.int32),
            pltpu.VMEM((_BPW,), jnp.int32),
            pltpu.VMEM((_BPW,), jnp.float32),
            pltpu.VMEM((_BPW,), jnp.float32),
            pltpu.VMEM((16,), jnp.float32),
            pltpu.VMEM((_CH, 2 * _EMB), jnp.float32),
            pltpu.VMEM((_CH, 2 * _EMB), jnp.float32),
            pltpu.SemaphoreType.DMA,
            pltpu.SemaphoreType.DMA,
            pltpu.SemaphoreType.DMA,
        ],
    )
    return fn(user_idx, item_idx, emb2, ubias1d, ibias1d, kvec)


def _tc_body(u_ref, i_ref, g_ref, t_ref, cn_ref, wide_ref, bias_ref,
             w1u_ref, w1i_ref, w1g_ref, w1t_ref, w1c_ref, c1_ref,
             w2_ref, c2_ref, wout_ref, wrow_ref, out_ref):
    f32 = jnp.float32
    tdot = lambda a, b: lax.dot_general(
        a, b, (((0,), (0,)), ((), ())), preferred_element_type=f32)
    h1 = jnp.dot(u_ref[...], w1u_ref[...], preferred_element_type=f32)
    h1 = h1 + jnp.dot(i_ref[...], w1i_ref[...], preferred_element_type=f32)
    h1 = h1 + tdot(g_ref[...], w1g_ref[...])
    h1 = h1 + tdot(t_ref[...], w1t_ref[...])
    h1 = h1 + tdot(cn_ref[...], w1c_ref[...])
    h1 = jnp.maximum(h1 + c1_ref[...], 0.0)
    h2 = jnp.dot(h1, w2_ref[...], preferred_element_type=f32)
    h2 = jnp.maximum(h2 + c2_ref[...], 0.0)
    deep = jnp.dot(h2, wout_ref[...], preferred_element_type=f32)
    wide = tdot(wide_ref[...], wrow_ref[...])
    out_ref[...] = bias_ref[...] + deep[:, 0] + wide[:, 0]


def _tc_dense(u, i, genre, tag, cont, wide_features, bias,
              w1u, w1i, w1g, w1t, w1c, c1, w2, c2, woutT, wrowT):
    grid = (_B // _BLK,)
    return pl.pallas_call(
        _tc_body,
        grid=grid,
        in_specs=[
            pl.BlockSpec((_BLK, 2 * _EMB), lambda i: (i, 0)),
            pl.BlockSpec((_BLK, 2 * _EMB), lambda i: (i, 0)),
            pl.BlockSpec((20, _BLK), lambda i: (0, i)),
            pl.BlockSpec((100, _BLK), lambda i: (0, i)),
            pl.BlockSpec((5, _BLK), lambda i: (0, i)),
            pl.BlockSpec((36, _BLK), lambda i: (0, i)),
            pl.BlockSpec((_BLK,), lambda i: (i,)),
            pl.BlockSpec((2 * _EMB, 256), lambda i: (0, 0)),
            pl.BlockSpec((2 * _EMB, 256), lambda i: (0, 0)),
            pl.BlockSpec((20, 256), lambda i: (0, 0)),
            pl.BlockSpec((100, 256), lambda i: (0, 0)),
            pl.BlockSpec((5, 256), lambda i: (0, 0)),
            pl.BlockSpec((1, 256), lambda i: (0, 0)),
            pl.BlockSpec((256, 128), lambda i: (0, 0)),
            pl.BlockSpec((1, 128), lambda i: (0, 0)),
            pl.BlockSpec((128, 1), lambda i: (0, 0)),
            pl.BlockSpec((36, 1), lambda i: (0, 0)),
        ],
        out_specs=pl.BlockSpec((_BLK,), lambda i: (i,)),
        out_shape=jax.ShapeDtypeStruct((_B,), jnp.float32),
        compiler_params=pltpu.CompilerParams(
            dimension_semantics=("arbitrary",),
        ),
    )(u, i, genre, tag, cont, wide_features, bias,
      w1u, w1i, w1g, w1t, w1c, c1, w2, c2, woutT, wrowT)


def kernel(user_idx, item_idx, genre, tag, wide_features, deep_continuous,
           user_bias, item_bias, user_emb, item_emb, wide_W, wide_b,
           W1, b1, g1, be1, W2, b2, g2, be2, Wout, bout):
    inv = np.float32(1.0 / np.sqrt(1.0 + _BN_EPS))
    s1 = g1 * inv
    s2 = g2 * inv
    w1f = (W1 * s1[:, None]).T
    c1 = (b1 * s1 + be1)[None, :]
    w2f = (W2 * s2[:, None]).T
    c2 = (b2 * s2 + be2)[None, :]
    zpad = jnp.zeros((_EMB, 256), jnp.float32)
    w1u = jnp.concatenate([w1f[:_EMB], zpad], axis=0)
    w1i = jnp.concatenate([zpad, w1f[_EMB:2 * _EMB]], axis=0)
    w1g = w1f[128:148]
    w1t = w1f[148:248]
    w1c = w1f[248:253]
    kvec = jnp.broadcast_to(
        wide_b[0] + bout[0] + jnp.float32(_GLOBAL_MEAN), (16,))
    emb2 = jnp.concatenate([user_emb, item_emb], axis=1)
    u, i, bias = _sc_gather(user_idx, item_idx, emb2,
                            user_bias[:, 0], item_bias[:, 0], kvec)
    return _tc_dense(u, i, genre.T, tag.T, deep_continuous.T,
                     wide_features.T, bias,
                     w1u, w1i, w1g, w1t, w1c, c1, w2f, c2,
                     Wout.T, wide_W.T)


# speedup vs baseline: 1.8593x; 1.0059x over previous
import jax
import jax.numpy as jnp
import numpy as np
from jax import lax
from jax.experimental import pallas as pl
from jax.experimental.pallas import tpu as pltpu
from jax.experimental.pallas import tpu_sc as plsc

_B = 16384
_EMB = 64
_BN_EPS = 1e-5
_GLOBAL_MEAN = 3.5

_NC = 2
_NS = 16
_NW = _NC * _NS
_BPW = _B // _NW
_CH = 128

_BLK = 2048


def _sc_gather_body(uidx_hbm, iidx_hbm, emb2_hbm, ubias_hbm, ibias_hbm,
                    kvec_hbm,
                    u_out, i_out, bias_out,
                    uraw_v, iraw_v, ub_v, ib_v, kv_v, p0_v, p1_v,
                    sem0, sem1, semb):
    wid = lax.axis_index("s") * _NC + lax.axis_index("c")
    base = wid * _BPW
    pltpu.sync_copy(uidx_hbm.at[pl.ds(base, _BPW)], uraw_v)
    pltpu.sync_copy(iidx_hbm.at[pl.ds(base, _BPW)], iraw_v)
    pltpu.sync_copy(kvec_hbm, kv_v)
    cb_u = pltpu.async_copy(ubias_hbm.at[uraw_v], ub_v, semb)
    cb_i = pltpu.async_copy(ibias_hbm.at[iraw_v], ib_v, semb)

    bufs = (p0_v, p1_v)
    sems = (sem0, sem1)
    n_c = _BPW // _CH
    jobs = [(uraw_v, u_out, c) for c in range(n_c)] + \
           [(iraw_v, i_out, c) for c in range(n_c)]

    inflight = [None, None]
    for j, (qref, out_hbm, c) in enumerate(jobs):
        slot = j % 2
        if inflight[slot] is not None:
            inflight[slot].wait()
            prev_out, prev_c = jobs[j - 2][1], jobs[j - 2][2]
            pltpu.sync_copy(bufs[slot],
                            prev_out.at[pl.ds(base + prev_c * _CH, _CH), :])
        inflight[slot] = pltpu.async_copy(
            emb2_hbm.at[qref.at[pl.ds(c * _CH, _CH)]], bufs[slot], sems[slot])
    for j in (len(jobs) - 2, len(jobs) - 1):
        slot = j % 2
        inflight[slot].wait()
        out_hbm, c = jobs[j][1], jobs[j][2]
        pltpu.sync_copy(bufs[slot],
                        out_hbm.at[pl.ds(base + c * _CH, _CH), :])

    cb_u.wait()
    cb_i.wait()
    kv = kv_v[...]
    for j in range(_BPW // 16):
        sl = pl.ds(j * 16, 16)
        ub_v[sl] = ub_v[sl] + ib_v[sl] + kv
    pltpu.sync_copy(ub_v, bias_out.at[pl.ds(base, _BPW)])


def _sc_gather(user_idx, item_idx, emb2, ubias1d, ibias1d, kvec):
    fn = pl.kernel(
        _sc_gather_body,
        out_type=[
            jax.ShapeDtypeStruct((_B, 2 * _EMB), jnp.float32),
            jax.ShapeDtypeStruct((_B, 2 * _EMB), jnp.float32),
            jax.ShapeDtypeStruct((_B,), jnp.float32),
        ],
        mesh=plsc.VectorSubcoreMesh(core_axis_name="c", subcore_axis_name="s"),
        scratch_types=[
            pltpu.VMEM((_BPW,), jnp.int32),
            pltpu.VMEM((_BPW,), jnp.int32),
            pltpu.VMEM((_BPW,), jnp.float32),
            pltpu.VMEM((_BPW,), jnp.float32),
            pltpu.VMEM((16,), jnp.float32),
            pltpu.VMEM((_CH, 2 * _EMB), jnp.float32),
            pltpu.VMEM((_CH, 2 * _EMB), jnp.float32),
            pltpu.SemaphoreType.DMA,
            pltpu.SemaphoreType.DMA,
            pltpu.SemaphoreType.DMA,
        ],
    )
    return fn(user_idx, item_idx, emb2, ubias1d, ibias1d, kvec)


def _tc_body(u_ref, i_ref, f_ref, wide_ref, bias_ref,
             w1u_ref, w1i_ref, w1ftc_ref, c1_ref,
             w2_ref, c2_ref, wout_ref, wrow_ref, out_ref):
    f32 = jnp.float32
    tdot = lambda a, b: lax.dot_general(
        a, b, (((0,), (0,)), ((), ())), preferred_element_type=f32)
    h1 = jnp.dot(u_ref[...], w1u_ref[...], preferred_element_type=f32)
    h1 = h1 + jnp.dot(i_ref[...], w1i_ref[...], preferred_element_type=f32)
    h1 = h1 + tdot(f_ref[...], w1ftc_ref[...])
    h1 = jnp.maximum(h1 + c1_ref[...], 0.0)
    h2 = jnp.dot(h1, w2_ref[...], preferred_element_type=f32)
    h2 = jnp.maximum(h2 + c2_ref[...], 0.0)
    deep = jnp.dot(h2, wout_ref[...], preferred_element_type=f32)
    wide = tdot(wide_ref[...], wrow_ref[...])
    out_ref[...] = bias_ref[...] + deep[:, 0] + wide[:, 0]


def _tc_dense(u, i, feats, wide_features, bias,
              w1u, w1i, w1ftc, c1, w2, c2, woutT, wrowT):
    grid = (_B // _BLK,)
    return pl.pallas_call(
        _tc_body,
        grid=grid,
        in_specs=[
            pl.BlockSpec((_BLK, 2 * _EMB), lambda i: (i, 0)),
            pl.BlockSpec((_BLK, 2 * _EMB), lambda i: (i, 0)),
            pl.BlockSpec((125, _BLK), lambda i: (0, i)),
            pl.BlockSpec((36, _BLK), lambda i: (0, i)),
            pl.BlockSpec((_BLK,), lambda i: (i,)),
            pl.BlockSpec((2 * _EMB, 256), lambda i: (0, 0)),
            pl.BlockSpec((2 * _EMB, 256), lambda i: (0, 0)),
            pl.BlockSpec((125, 256), lambda i: (0, 0)),
            pl.BlockSpec((1, 256), lambda i: (0, 0)),
            pl.BlockSpec((256, 128), lambda i: (0, 0)),
            pl.BlockSpec((1, 128), lambda i: (0, 0)),
            pl.BlockSpec((128, 1), lambda i: (0, 0)),
            pl.BlockSpec((36, 1), lambda i: (0, 0)),
        ],
        out_specs=pl.BlockSpec((_BLK,), lambda i: (i,)),
        out_shape=jax.ShapeDtypeStruct((_B,), jnp.float32),
        compiler_params=pltpu.CompilerParams(
            dimension_semantics=("arbitrary",),
        ),
    )(u, i, feats, wide_features, bias,
      w1u, w1i, w1ftc, c1, w2, c2, woutT, wrowT)


def kernel(user_idx, item_idx, genre, tag, wide_features, deep_continuous,
           user_bias, item_bias, user_emb, item_emb, wide_W, wide_b,
           W1, b1, g1, be1, W2, b2, g2, be2, Wout, bout):
    inv = np.float32(1.0 / np.sqrt(1.0 + _BN_EPS))
    s1 = g1 * inv
    s2 = g2 * inv
    w1f = (W1 * s1[:, None]).T
    c1 = (b1 * s1 + be1)[None, :]
    w2f = (W2 * s2[:, None]).T
    c2 = (b2 * s2 + be2)[None, :]
    zpad = jnp.zeros((_EMB, 256), jnp.float32)
    w1u = jnp.concatenate([w1f[:_EMB], zpad], axis=0)
    w1i = jnp.concatenate([zpad, w1f[_EMB:2 * _EMB]], axis=0)
    w1ftc = w1f[128:253]
    kvec = jnp.broadcast_to(
        wide_b[0] + bout[0] + jnp.float32(_GLOBAL_MEAN), (16,))
    emb2 = jnp.concatenate([user_emb, item_emb], axis=1)
    feats = jnp.concatenate([genre, tag, deep_continuous], axis=1).T
    u, i, bias = _sc_gather(user_idx, item_idx, emb2,
                            user_bias[:, 0], item_bias[:, 0], kvec)
    return _tc_dense(u, i, feats, wide_features.T, bias,
                     w1u, w1i, w1ftc, c1, w2f, c2,
                     Wout.T, wide_W.T)


# speedup vs baseline: 1.8615x; 1.0012x over previous
import jax
import jax.numpy as jnp
import numpy as np
from jax import lax
from jax.experimental import pallas as pl
from jax.experimental.pallas import tpu as pltpu
from jax.experimental.pallas import tpu_sc as plsc

_B = 16384
_EMB = 64
_BN_EPS = 1e-5
_GLOBAL_MEAN = 3.5

_NC = 2
_NS = 16
_NW = _NC * _NS
_BPW = _B // _NW
_CH = 128

_BLK = 4096


def _sc_gather_body(uidx_hbm, iidx_hbm, emb2_hbm, ubias_hbm, ibias_hbm,
                    kvec_hbm,
                    u_out, i_out, bias_out,
                    uraw_v, iraw_v, ub_v, ib_v, kv_v, p0_v, p1_v,
                    sem0, sem1, semb):
    wid = lax.axis_index("s") * _NC + lax.axis_index("c")
    base = wid * _BPW
    pltpu.sync_copy(uidx_hbm.at[pl.ds(base, _BPW)], uraw_v)
    pltpu.sync_copy(iidx_hbm.at[pl.ds(base, _BPW)], iraw_v)
    pltpu.sync_copy(kvec_hbm, kv_v)
    cb_u = pltpu.async_copy(ubias_hbm.at[uraw_v], ub_v, semb)
    cb_i = pltpu.async_copy(ibias_hbm.at[iraw_v], ib_v, semb)

    bufs = (p0_v, p1_v)
    sems = (sem0, sem1)
    n_c = _BPW // _CH
    jobs = [(uraw_v, u_out, c) for c in range(n_c)] + \
           [(iraw_v, i_out, c) for c in range(n_c)]

    inflight = [None, None]
    for j, (qref, out_hbm, c) in enumerate(jobs):
        slot = j % 2
        if inflight[slot] is not None:
            inflight[slot].wait()
            prev_out, prev_c = jobs[j - 2][1], jobs[j - 2][2]
            pltpu.sync_copy(bufs[slot],
                            prev_out.at[pl.ds(base + prev_c * _CH, _CH), :])
        inflight[slot] = pltpu.async_copy(
            emb2_hbm.at[qref.at[pl.ds(c * _CH, _CH)]], bufs[slot], sems[slot])
    for j in (len(jobs) - 2, len(jobs) - 1):
        slot = j % 2
        inflight[slot].wait()
        out_hbm, c = jobs[j][1], jobs[j][2]
        pltpu.sync_copy(bufs[slot],
                        out_hbm.at[pl.ds(base + c * _CH, _CH), :])

    cb_u.wait()
    cb_i.wait()
    kv = kv_v[...]
    for j in range(_BPW // 16):
        sl = pl.ds(j * 16, 16)
        ub_v[sl] = ub_v[sl] + ib_v[sl] + kv
    pltpu.sync_copy(ub_v, bias_out.at[pl.ds(base, _BPW)])


def _sc_gather(user_idx, item_idx, emb2, ubias1d, ibias1d, kvec):
    fn = pl.kernel(
        _sc_gather_body,
        out_type=[
            jax.ShapeDtypeStruct((_B, 2 * _EMB), jnp.float32),
            jax.ShapeDtypeStruct((_B, 2 * _EMB), jnp.float32),
            jax.ShapeDtypeStruct((_B,), jnp.float32),
        ],
        mesh=plsc.VectorSubcoreMesh(core_axis_name="c", subcore_axis_name="s"),
        scratch_types=[
            pltpu.VMEM((_BPW,), jnp.int32),
            pltpu.VMEM((_BPW,), jnp.int32),
            pltpu.VMEM((_BPW,), jnp.float32),
            pltpu.VMEM((_BPW,), jnp.float32),
            pltpu.VMEM((16,), jnp.float32),
            pltpu.VMEM((_CH, 2 * _EMB), jnp.float32),
            pltpu.VMEM((_CH, 2 * _EMB), jnp.float32),
            pltpu.SemaphoreType.DMA,
            pltpu.SemaphoreType.DMA,
            pltpu.SemaphoreType.DMA,
        ],
    )
    return fn(user_idx, item_idx, emb2, ubias1d, ibias1d, kvec)


def _tc_body(u_ref, i_ref, f_ref, wide_ref, bias_ref,
             w1u_ref, w1i_ref, w1ftc_ref, c1_ref,
             w2_ref, c2_ref, wout_ref, wrow_ref, out_ref):
    f32 = jnp.float32
    tdot = lambda a, b: lax.dot_general(
        a, b, (((0,), (0,)), ((), ())), preferred_element_type=f32)
    h1 = jnp.dot(u_ref[...], w1u_ref[...], preferred_element_type=f32)
    h1 = h1 + jnp.dot(i_ref[...], w1i_ref[...], preferred_element_type=f32)
    h1 = h1 + tdot(f_ref[...], w1ftc_ref[...])
    h1 = jnp.maximum(h1 + c1_ref[...], 0.0)
    h2 = jnp.dot(h1, w2_ref[...], preferred_element_type=f32)
    h2 = jnp.maximum(h2 + c2_ref[...], 0.0)
    deep = jnp.dot(h2, wout_ref[...], preferred_element_type=f32)
    wide = tdot(wide_ref[...], wrow_ref[...])
    out_ref[...] = bias_ref[...] + deep[:, 0] + wide[:, 0]


def _tc_dense(u, i, feats, wide_features, bias,
              w1u, w1i, w1ftc, c1, w2, c2, woutT, wrowT):
    grid = (_B // _BLK,)
    return pl.pallas_call(
        _tc_body,
        grid=grid,
        in_specs=[
            pl.BlockSpec((_BLK, 2 * _EMB), lambda i: (i, 0)),
            pl.BlockSpec((_BLK, 2 * _EMB), lambda i: (i, 0)),
            pl.BlockSpec((125, _BLK), lambda i: (0, i)),
            pl.BlockSpec((36, _BLK), lambda i: (0, i)),
            pl.BlockSpec((_BLK,), lambda i: (i,)),
            pl.BlockSpec((2 * _EMB, 256), lambda i: (0, 0)),
            pl.BlockSpec((2 * _EMB, 256), lambda i: (0, 0)),
            pl.BlockSpec((125, 256), lambda i: (0, 0)),
            pl.BlockSpec((1, 256), lambda i: (0, 0)),
            pl.BlockSpec((256, 128), lambda i: (0, 0)),
            pl.BlockSpec((1, 128), lambda i: (0, 0)),
            pl.BlockSpec((128, 1), lambda i: (0, 0)),
            pl.BlockSpec((36, 1), lambda i: (0, 0)),
        ],
        out_specs=pl.BlockSpec((_BLK,), lambda i: (i,)),
        out_shape=jax.ShapeDtypeStruct((_B,), jnp.float32),
        compiler_params=pltpu.CompilerParams(
            dimension_semantics=("arbitrary",),
        ),
    )(u, i, feats, wide_features, bias,
      w1u, w1i, w1ftc, c1, w2, c2, woutT, wrowT)


def kernel(user_idx, item_idx, genre, tag, wide_features, deep_continuous,
           user_bias, item_bias, user_emb, item_emb, wide_W, wide_b,
           W1, b1, g1, be1, W2, b2, g2, be2, Wout, bout):
    inv = np.float32(1.0 / np.sqrt(1.0 + _BN_EPS))
    s1 = g1 * inv
    s2 = g2 * inv
    w1f = (W1 * s1[:, None]).T
    c1 = (b1 * s1 + be1)[None, :]
    w2f = (W2 * s2[:, None]).T
    c2 = (b2 * s2 + be2)[None, :]
    zpad = jnp.zeros((_EMB, 256), jnp.float32)
    w1u = jnp.concatenate([w1f[:_EMB], zpad], axis=0)
    w1i = jnp.concatenate([zpad, w1f[_EMB:2 * _EMB]], axis=0)
    w1ftc = w1f[128:253]
    kvec = jnp.broadcast_to(
        wide_b[0] + bout[0] + jnp.float32(_GLOBAL_MEAN), (16,))
    emb2 = jnp.concatenate([user_emb, item_emb], axis=1)
    feats = jnp.concatenate([genre, tag, deep_continuous], axis=1).T
    u, i, bias = _sc_gather(user_idx, item_idx, emb2,
                            user_bias[:, 0], item_bias[:, 0], kvec)
    return _tc_dense(u, i, feats, wide_features.T, bias,
                     w1u, w1i, w1ftc, c1, w2f, c2,
                     Wout.T, wide_W.T)
